# Initial kernel scaffold; baseline (speedup 1.0000x reference)
import jax
import jax.numpy as jnp
from jax.experimental import pallas as pl


def kernel(centers, contexts, negatives, in_embed, out_embed):
    raise NotImplementedError("write your pallas kernel here")


# speedup vs baseline: 4.2447x; 4.2447x over previous
import functools

import jax
import jax.numpy as jnp
from jax import lax
from jax.experimental import pallas as pl
from jax.experimental.pallas import tpu as pltpu
from jax.experimental.pallas import tpu_sc as plsc

_VOCAB = 1000000
_D = 64
_B = 16384
_K = 20
_NC = 2
_NS = 16
_NW = _NC * _NS
_BPW = _B // _NW
_CH = 32
_NR = _BPW // _CH
_NEG_CH = _CH * _K
_NIDX_ROWS = _NEG_CH // 128


def _sc_scores(cen, ctx, neg, in_embed, out_embed):
    mesh = plsc.VectorSubcoreMesh(core_axis_name="c", subcore_axis_name="s")

    @functools.partial(
        pl.kernel,
        mesh=mesh,
        compiler_params=pltpu.CompilerParams(use_tc_tiling_on_sc=False),
        out_type=[
            jax.ShapeDtypeStruct((_B * 16,), jnp.float32),
            jax.ShapeDtypeStruct((_B * _K * 16,), jnp.float32),
        ],
        scratch_types=[
            pltpu.VMEM((_CH,), jnp.int32),
            pltpu.VMEM((_CH,), jnp.int32),
            pltpu.VMEM((_NEG_CH,), jnp.int32),
            pltpu.VMEM((_CH, _D), jnp.float32),
            pltpu.VMEM((_CH, _D), jnp.float32),
            pltpu.VMEM((_NEG_CH, _D), jnp.float32),
            pltpu.VMEM((_CH * 16,), jnp.float32),
            pltpu.VMEM((_NEG_CH * 16,), jnp.float32),
            pltpu.SemaphoreType.DMA,
        ],
    )
    def k(cen_hbm, ctx_hbm, neg_hbm, ine_hbm, oute_hbm,
          pos_out, neg_out,
          cen_i, ctx_i, neg_i, cen_r, ctx_r, neg_r, pos_p, neg_p, sem):
        wid = lax.axis_index("s") * _NC + lax.axis_index("c")
        base = wid * _BPW

        def round_body(r, carry):
            b0 = base + r * _CH
            pltpu.sync_copy(cen_hbm.at[pl.ds(b0, _CH)], cen_i)
            pltpu.sync_copy(ctx_hbm.at[pl.ds(b0, _CH)], ctx_i)
            pltpu.sync_copy(neg_hbm.at[pl.ds(b0 * _K, _NEG_CH)], neg_i)
            cp1 = pltpu.async_copy(ine_hbm.at[cen_i], cen_r, sem)
            cp2 = pltpu.async_copy(oute_hbm.at[ctx_i], ctx_r, sem)
            cps = [
                pltpu.async_copy(oute_hbm.at[neg_i.at[pl.ds(j * 128, 128)]],
                                 neg_r.at[pl.ds(j * 128, 128)], sem)
                for j in range(_NIDX_ROWS)
            ]
            cp1.wait()
            cp2.wait()
            for cp in cps:
                cp.wait()

            def b_body(b, carry2):
                c0 = cen_r[b, pl.ds(0, 16)]
                c1 = cen_r[b, pl.ds(16, 16)]
                c2 = cen_r[b, pl.ds(32, 16)]
                c3 = cen_r[b, pl.ds(48, 16)]
                x0 = ctx_r[b, pl.ds(0, 16)]
                x1 = ctx_r[b, pl.ds(16, 16)]
                x2 = ctx_r[b, pl.ds(32, 16)]
                x3 = ctx_r[b, pl.ds(48, 16)]
                pos_p[pl.ds(b * 16, 16)] = c0 * x0 + c1 * x1 + c2 * x2 + c3 * x3
                nb = b * _K
                for kk in range(_K):
                    n0 = neg_r[nb + kk, pl.ds(0, 16)]
                    n1 = neg_r[nb + kk, pl.ds(16, 16)]
                    n2 = neg_r[nb + kk, pl.ds(32, 16)]
                    n3 = neg_r[nb + kk, pl.ds(48, 16)]
                    neg_p[pl.ds((nb + kk) * 16, 16)] = (
                        c0 * n0 + c1 * n1 + c2 * n2 + c3 * n3)
                return carry2

            lax.fori_loop(0, _CH, b_body, 0)
            cpo1 = pltpu.async_copy(pos_p, pos_out.at[pl.ds(b0 * 16, _CH * 16)],
                                    sem)
            cpo2 = pltpu.async_copy(
                neg_p, neg_out.at[pl.ds(b0 * _K * 16, _NEG_CH * 16)], sem)
            cpo1.wait()
            cpo2.wait()
            return carry

        lax.fori_loop(0, _NR, round_body, 0)

    return k(cen, ctx, neg, in_embed, out_embed)


def _tc_loss(pos_p, neg_p):
    grid = 16
    pos_rows = _B // grid
    neg_rows = _B * _K // grid

    def body(p_ref, n_ref, o_ref):
        ps = jnp.sum(p_ref[...], axis=1)
        ns = jnp.sum(n_ref[...], axis=1)
        lsp = jnp.minimum(ps, 0.0) - jnp.log1p(jnp.exp(-jnp.abs(ps)))
        lsn = jnp.minimum(-ns, 0.0) - jnp.log1p(jnp.exp(-jnp.abs(ns)))
        partial = jnp.sum(lsp) + jnp.sum(lsn)

        @pl.when(pl.program_id(0) == 0)
        def _():
            o_ref[...] = jnp.zeros((1, 1), jnp.float32)

        o_ref[...] += jnp.reshape(-partial / _B, (1, 1))

    out = pl.pallas_call(
        body,
        grid=(grid,),
        in_specs=[
            pl.BlockSpec((pos_rows, 16), lambda i: (i, 0)),
            pl.BlockSpec((neg_rows, 16), lambda i: (i, 0)),
        ],
        out_specs=pl.BlockSpec((1, 1), lambda i: (0, 0)),
        out_shape=jax.ShapeDtypeStruct((1, 1), jnp.float32),
    )(pos_p.reshape(_B, 16), neg_p.reshape(_B * _K, 16))
    return out[0, 0]


def kernel(centers, contexts, negatives, in_embed, out_embed):
    cen = centers.astype(jnp.int32)
    ctx = contexts.astype(jnp.int32)
    neg = negatives.astype(jnp.int32).reshape(_B * _K)
    pos_p, neg_p = _sc_scores(cen, ctx, neg, in_embed, out_embed)
    return _tc_loss(pos_p, neg_p)
